# initial kernel scaffold (unmeasured)
import jax
import jax.numpy as jnp
from jax import lax
from jax.experimental import pallas as pl
from jax.experimental.pallas import tpu as pltpu

N_DEV = 4


def kernel(O, Wo):
    B, S, Hl, D = O.shape
    K = Hl * D
    N = Wo.shape[1]
    S_out = S // N_DEV
    NH = N // 2

    def body(o_hbm, w_ref, out_hbm, o_chunk, comm, load_sem, store_sem,
             send_sems, recv_sems):
        me = lax.axis_index("i")
        left = (me - 1) % N_DEV
        right = (me + 1) % N_DEV

        barrier = pltpu.get_barrier_semaphore()
        for nbr in (left, right):
            pl.semaphore_signal(barrier, inc=1, device_id=(nbr,),
                                device_id_type=pl.DeviceIdType.MESH)
        pl.semaphore_wait(barrier, 2)

        def load(c):
            cp = pltpu.make_async_copy(
                o_hbm.at[:, pl.ds(c * S_out, S_out)], o_chunk, load_sem)
            cp.start()
            cp.wait()

        def accum(slot, init):
            for b in range(B):
                lhs = o_chunk[b].reshape(S_out, K)
                for h in range(2):
                    p = jnp.dot(lhs, w_ref[:, h * NH:(h + 1) * NH],
                                preferred_element_type=jnp.float32)
                    if init:
                        comm[slot, b, :, h * NH:(h + 1) * NH] = p
                    else:
                        comm[slot, b, :, h * NH:(h + 1) * NH] += p

        load((me - 1) % N_DEV)
        accum(0, init=True)

        for t in range(N_DEV - 1):
            s_slot = t % 2
            r_slot = (t + 1) % 2
            rdma = pltpu.make_async_remote_copy(
                src_ref=comm.at[s_slot],
                dst_ref=comm.at[r_slot],
                send_sem=send_sems.at[t],
                recv_sem=recv_sems.at[t],
                device_id=(right,),
                device_id_type=pl.DeviceIdType.MESH,
            )
            rdma.start()
            load((me - t - 2) % N_DEV)
            rdma.wait()
            accum(r_slot, init=False)

        cp = pltpu.make_async_copy(comm.at[(N_DEV - 1) % 2], out_hbm, store_sem)
        cp.start()
        cp.wait()

    return pl.pallas_call(
        body,
        out_shape=jax.ShapeDtypeStruct((B, S_out, N), jnp.float32),
        in_specs=[pl.BlockSpec(memory_space=pltpu.ANY),
                  pl.BlockSpec(memory_space=pltpu.VMEM)],
        out_specs=pl.BlockSpec(memory_space=pltpu.ANY),
        scratch_shapes=[
            pltpu.VMEM((B, S_out, Hl, D), jnp.float32),
            pltpu.VMEM((2, B, S_out, N), jnp.float32),
            pltpu.SemaphoreType.DMA,
            pltpu.SemaphoreType.DMA,
            pltpu.SemaphoreType.DMA((N_DEV - 1,)),
            pltpu.SemaphoreType.DMA((N_DEV - 1,)),
        ],
        compiler_params=pltpu.CompilerParams(collective_id=0),
    )(O, Wo)


# baseline (device time: 630266 ns/iter reference)
import jax
import jax.numpy as jnp
from jax import lax
from jax.experimental import pallas as pl
from jax.experimental.pallas import tpu as pltpu

N_DEV = 4


def kernel(O, Wo):
    B, S, Hl, D = O.shape
    K = Hl * D
    N = Wo.shape[1]
    S_out = S // N_DEV
    NT = N // 4
    O = O.reshape(B, S, K).astype(jnp.bfloat16)
    Wo = Wo.astype(jnp.bfloat16)

    def body(o_hbm, w_ref, out_hbm, o_chunk, comm, load_sem, store_sem,
             send_sems, recv_sems):
        me = lax.axis_index("i")
        left = (me - 1) % N_DEV
        right = (me + 1) % N_DEV

        barrier = pltpu.get_barrier_semaphore()
        for nbr in (left, right):
            pl.semaphore_signal(barrier, inc=1, device_id=(nbr,),
                                device_id_type=pl.DeviceIdType.MESH)
        pl.semaphore_wait(barrier, 2)

        def load(c):
            cp = pltpu.make_async_copy(
                o_hbm.at[:, pl.ds(c * S_out, S_out)], o_chunk, load_sem)
            cp.start()
            cp.wait()

        def accum(slot, init):
            for b in range(B):
                lhs = o_chunk[b]
                for h in range(N // NT):
                    p = jnp.dot(lhs, w_ref[:, h * NT:(h + 1) * NT],
                                preferred_element_type=jnp.float32)
                    if init:
                        comm[slot, b, :, h * NT:(h + 1) * NT] = p
                    else:
                        comm[slot, b, :, h * NT:(h + 1) * NT] += p

        load((me - 1) % N_DEV)
        accum(0, init=True)

        for t in range(N_DEV - 1):
            s_slot = t % 2
            r_slot = (t + 1) % 2
            rdma = pltpu.make_async_remote_copy(
                src_ref=comm.at[s_slot],
                dst_ref=comm.at[r_slot],
                send_sem=send_sems.at[t],
                recv_sem=recv_sems.at[t],
                device_id=(right,),
                device_id_type=pl.DeviceIdType.MESH,
            )
            rdma.start()
            load((me - t - 2) % N_DEV)
            rdma.wait()
            accum(r_slot, init=False)

        cp = pltpu.make_async_copy(comm.at[(N_DEV - 1) % 2], out_hbm, store_sem)
        cp.start()
        cp.wait()

    return pl.pallas_call(
        body,
        out_shape=jax.ShapeDtypeStruct((B, S_out, N), jnp.float32),
        in_specs=[pl.BlockSpec(memory_space=pl.ANY),
                  pl.BlockSpec(memory_space=pltpu.VMEM)],
        out_specs=pl.BlockSpec(memory_space=pl.ANY),
        scratch_shapes=[
            pltpu.VMEM((B, S_out, K), jnp.bfloat16),
            pltpu.VMEM((2, B, S_out, N), jnp.float32),
            pltpu.SemaphoreType.DMA,
            pltpu.SemaphoreType.DMA,
            pltpu.SemaphoreType.DMA((N_DEV - 1,)),
            pltpu.SemaphoreType.DMA((N_DEV - 1,)),
        ],
        compiler_params=pltpu.CompilerParams(
            collective_id=0,
            vmem_limit_bytes=46 * 1024 * 1024,
        ),
    )(O, Wo)


# device time: 363114 ns/iter; 1.7357x vs baseline; 1.7357x over previous
import jax
import jax.numpy as jnp
from jax import lax
from jax.experimental import pallas as pl
from jax.experimental.pallas import tpu as pltpu

N_DEV = 4


def kernel(O, Wo):
    B, S, Hl, D = O.shape
    K = Hl * D
    N = Wo.shape[1]
    S_out = S // N_DEV
    NH = N // 2
    NT = N // 4
    O = O.reshape(B, S, K).astype(jnp.bfloat16)
    Wo = Wo.astype(jnp.bfloat16)

    def body(o_hbm, w_ref, out_hbm,
             oc_f, oc_r, comm_f, comm_r,
             load_f_sem, load_r_sem, store_sems,
             send_f, recv_f, send_r, recv_r):
        me = lax.axis_index("i")
        left = (me - 1) % N_DEV
        right = (me + 1) % N_DEV

        barrier = pltpu.get_barrier_semaphore()
        for nbr in (left, right):
            pl.semaphore_signal(barrier, inc=1, device_id=(nbr,),
                                device_id_type=pl.DeviceIdType.MESH)
        pl.semaphore_wait(barrier, 2)

        def load(o_ref, c, sem):
            cp = pltpu.make_async_copy(
                o_hbm.at[:, pl.ds(c * S_out, S_out)], o_ref, sem)
            cp.start()
            return cp

        def accum(comm, slot, o_ref, col0, init):
            for b in range(B):
                lhs = o_ref[b]
                for j in range(NH // NT):
                    p = jnp.dot(lhs, w_ref[:, col0 + j * NT:col0 + (j + 1) * NT],
                                preferred_element_type=jnp.float32)
                    if init:
                        comm[slot, b, :, j * NT:(j + 1) * NT] = p
                    else:
                        comm[slot, b, :, j * NT:(j + 1) * NT] += p

        load(oc_f, (me - 1) % N_DEV, load_f_sem).wait()
        accum(comm_f, 0, oc_f, 0, init=True)
        load(oc_r, (me + 1) % N_DEV, load_r_sem).wait()
        accum(comm_r, 0, oc_r, NH, init=True)

        for t in range(N_DEV - 1):
            s_slot = t % 2
            r_slot = (t + 1) % 2
            rdma_f = pltpu.make_async_remote_copy(
                src_ref=comm_f.at[s_slot], dst_ref=comm_f.at[r_slot],
                send_sem=send_f.at[t], recv_sem=recv_f.at[t],
                device_id=(right,), device_id_type=pl.DeviceIdType.MESH)
            rdma_r = pltpu.make_async_remote_copy(
                src_ref=comm_r.at[s_slot], dst_ref=comm_r.at[r_slot],
                send_sem=send_r.at[t], recv_sem=recv_r.at[t],
                device_id=(left,), device_id_type=pl.DeviceIdType.MESH)
            rdma_f.start()
            rdma_r.start()
            cp_f = load(oc_f, (me - t - 2) % N_DEV, load_f_sem)
            cp_r = load(oc_r, (me + t + 2) % N_DEV, load_r_sem)
            cp_f.wait()
            cp_r.wait()
            rdma_f.wait()
            accum(comm_f, r_slot, oc_f, 0, init=False)
            rdma_r.wait()
            accum(comm_r, r_slot, oc_r, NH, init=False)

        last = (N_DEV - 1) % 2
        cp0 = pltpu.make_async_copy(
            comm_f.at[last], out_hbm.at[:, :, pl.ds(0, NH)], store_sems.at[0])
        cp1 = pltpu.make_async_copy(
            comm_r.at[last], out_hbm.at[:, :, pl.ds(NH, NH)], store_sems.at[1])
        cp0.start()
        cp1.start()
        cp0.wait()
        cp1.wait()

    return pl.pallas_call(
        body,
        out_shape=jax.ShapeDtypeStruct((B, S_out, N), jnp.float32),
        in_specs=[pl.BlockSpec(memory_space=pl.ANY),
                  pl.BlockSpec(memory_space=pltpu.VMEM)],
        out_specs=pl.BlockSpec(memory_space=pl.ANY),
        scratch_shapes=[
            pltpu.VMEM((B, S_out, K), jnp.bfloat16),
            pltpu.VMEM((B, S_out, K), jnp.bfloat16),
            pltpu.VMEM((2, B, S_out, NH), jnp.float32),
            pltpu.VMEM((2, B, S_out, NH), jnp.float32),
            pltpu.SemaphoreType.DMA,
            pltpu.SemaphoreType.DMA,
            pltpu.SemaphoreType.DMA((2,)),
            pltpu.SemaphoreType.DMA((N_DEV - 1,)),
            pltpu.SemaphoreType.DMA((N_DEV - 1,)),
            pltpu.SemaphoreType.DMA((N_DEV - 1,)),
            pltpu.SemaphoreType.DMA((N_DEV - 1,)),
        ],
        compiler_params=pltpu.CompilerParams(
            collective_id=0,
            vmem_limit_bytes=46 * 1024 * 1024,
        ),
    )(O, Wo)


# device time: 228274 ns/iter; 2.7610x vs baseline; 1.5907x over previous
import jax
import jax.numpy as jnp
from jax import lax
from jax.experimental import pallas as pl
from jax.experimental.pallas import tpu as pltpu

N_DEV = 4


def kernel(O, Wo):
    B, S, Hl, D = O.shape
    K = Hl * D
    N = Wo.shape[1]
    S_out = S // N_DEV
    NH = N // 2
    NT = N // 4
    O = O.reshape(B, S, K).astype(jnp.bfloat16)
    Wo = Wo.astype(jnp.bfloat16)

    def body(o_hbm, w_ref, out_hbm,
             oc_f, oc_r, comm_f, comm_r, stage_f, stage_r,
             load_f_sem, load_r_sem, store_sems,
             send_f, recv_f, send_r, recv_r):
        me = lax.axis_index("i")
        left = (me - 1) % N_DEV
        right = (me + 1) % N_DEV

        barrier = pltpu.get_barrier_semaphore()
        for nbr in (left, right):
            pl.semaphore_signal(barrier, inc=1, device_id=(nbr,),
                                device_id_type=pl.DeviceIdType.MESH)
        pl.semaphore_wait(barrier, 2)

        def load(o_ref, c, sem):
            cp = pltpu.make_async_copy(
                o_hbm.at[:, pl.ds(c * S_out, S_out)], o_ref, sem)
            cp.start()
            return cp

        def accum(comm, slot, o_ref, col0, init, final_stage=None):
            for b in range(B):
                lhs = o_ref[b]
                for j in range(NH // NT):
                    p = jnp.dot(lhs, w_ref[:, col0 + j * NT:col0 + (j + 1) * NT],
                                preferred_element_type=jnp.float32)
                    tile = (b, slice(None), slice(j * NT, (j + 1) * NT))
                    if init:
                        comm[(slot, *tile)] = p.astype(jnp.bfloat16)
                    elif final_stage is not None:
                        final_stage[tile] = comm[(slot, *tile)].astype(
                            jnp.float32) + p
                    else:
                        comm[(slot, *tile)] = (comm[(slot, *tile)].astype(
                            jnp.float32) + p).astype(jnp.bfloat16)

        load(oc_f, (me - 1) % N_DEV, load_f_sem).wait()
        accum(comm_f, 0, oc_f, 0, init=True)
        load(oc_r, (me + 1) % N_DEV, load_r_sem).wait()
        accum(comm_r, 0, oc_r, NH, init=True)

        for t in range(N_DEV - 1):
            s_slot = t % 2
            r_slot = (t + 1) % 2
            rdma_f = pltpu.make_async_remote_copy(
                src_ref=comm_f.at[s_slot], dst_ref=comm_f.at[r_slot],
                send_sem=send_f.at[t], recv_sem=recv_f.at[t],
                device_id=(right,), device_id_type=pl.DeviceIdType.MESH)
            rdma_r = pltpu.make_async_remote_copy(
                src_ref=comm_r.at[s_slot], dst_ref=comm_r.at[r_slot],
                send_sem=send_r.at[t], recv_sem=recv_r.at[t],
                device_id=(left,), device_id_type=pl.DeviceIdType.MESH)
            rdma_f.start()
            rdma_r.start()
            cp_f = load(oc_f, (me - t - 2) % N_DEV, load_f_sem)
            cp_r = load(oc_r, (me + t + 2) % N_DEV, load_r_sem)
            cp_f.wait()
            cp_r.wait()
            final = t == N_DEV - 2
            rdma_f.wait()
            accum(comm_f, r_slot, oc_f, 0, init=False,
                  final_stage=stage_f if final else None)
            rdma_r.wait()
            accum(comm_r, r_slot, oc_r, NH, init=False,
                  final_stage=stage_r if final else None)

        cp0 = pltpu.make_async_copy(
            stage_f, out_hbm.at[:, :, pl.ds(0, NH)], store_sems.at[0])
        cp1 = pltpu.make_async_copy(
            stage_r, out_hbm.at[:, :, pl.ds(NH, NH)], store_sems.at[1])
        cp0.start()
        cp1.start()
        cp0.wait()
        cp1.wait()

    return pl.pallas_call(
        body,
        out_shape=jax.ShapeDtypeStruct((B, S_out, N), jnp.float32),
        in_specs=[pl.BlockSpec(memory_space=pl.ANY),
                  pl.BlockSpec(memory_space=pltpu.VMEM)],
        out_specs=pl.BlockSpec(memory_space=pl.ANY),
        scratch_shapes=[
            pltpu.VMEM((B, S_out, K), jnp.bfloat16),
            pltpu.VMEM((B, S_out, K), jnp.bfloat16),
            pltpu.VMEM((2, B, S_out, NH), jnp.bfloat16),
            pltpu.VMEM((2, B, S_out, NH), jnp.bfloat16),
            pltpu.VMEM((B, S_out, NH), jnp.float32),
            pltpu.VMEM((B, S_out, NH), jnp.float32),
            pltpu.SemaphoreType.DMA,
            pltpu.SemaphoreType.DMA,
            pltpu.SemaphoreType.DMA((2,)),
            pltpu.SemaphoreType.DMA((N_DEV - 1,)),
            pltpu.SemaphoreType.DMA((N_DEV - 1,)),
            pltpu.SemaphoreType.DMA((N_DEV - 1,)),
            pltpu.SemaphoreType.DMA((N_DEV - 1,)),
        ],
        compiler_params=pltpu.CompilerParams(
            collective_id=0,
            vmem_limit_bytes=46 * 1024 * 1024,
        ),
    )(O, Wo)


# device time: 192346 ns/iter; 3.2767x vs baseline; 1.1868x over previous
import jax
import jax.numpy as jnp
from jax import lax
from jax.experimental import pallas as pl
from jax.experimental.pallas import tpu as pltpu

N_DEV = 4
NSUB = 2


def kernel(O, Wo):
    B, S, Hl, D = O.shape
    K = Hl * D
    N = Wo.shape[1]
    S_out = S // N_DEV
    NH = N // 2
    SUB = NH // NSUB
    O = O.reshape(B, S, K).astype(jnp.bfloat16)
    Wo = Wo.astype(jnp.bfloat16)

    def body(o_hbm, w_ref, out_hbm,
             oc_f, oc_r, comm_f, comm_r, stage_f, stage_r,
             load_f_sem, load_r_sem, store_sems,
             send_f, recv_f, send_r, recv_r):
        me = lax.axis_index("i")
        left = (me - 1) % N_DEV
        right = (me + 1) % N_DEV

        barrier = pltpu.get_barrier_semaphore()
        for nbr in (left, right):
            pl.semaphore_signal(barrier, inc=1, device_id=(nbr,),
                                device_id_type=pl.DeviceIdType.MESH)
        pl.semaphore_wait(barrier, 2)

        def load(o_ref, c, sem):
            cp = pltpu.make_async_copy(
                o_hbm.at[:, pl.ds(c * S_out, S_out)], o_ref, sem)
            cp.start()
            return cp

        def start_hop(comm, h, u, ssem, rsem, dev):
            r = pltpu.make_async_remote_copy(
                src_ref=comm.at[h % 2, u],
                dst_ref=comm.at[(h + 1) % 2, u],
                send_sem=ssem.at[h, u], recv_sem=rsem.at[h, u],
                device_id=(dev,), device_id_type=pl.DeviceIdType.MESH)
            r.start()
            return r

        def accum(comm, slot, u, o_ref, col0, init=False, final_stage=None):
            for b in range(B):
                p = jnp.dot(o_ref[b],
                            w_ref[:, col0 + u * SUB:col0 + (u + 1) * SUB],
                            preferred_element_type=jnp.float32)
                if init:
                    comm[slot, u, b] = p.astype(jnp.bfloat16)
                elif final_stage is not None:
                    final_stage[b, :, u * SUB:(u + 1) * SUB] = (
                        comm[slot, u, b].astype(jnp.float32) + p)
                else:
                    comm[slot, u, b] = (
                        comm[slot, u, b].astype(jnp.float32) + p
                    ).astype(jnp.bfloat16)

        rd = {}
        cpf = load(oc_f, (me - 1) % N_DEV, load_f_sem)
        cpr = load(oc_r, (me + 1) % N_DEV, load_r_sem)
        cpf.wait()
        cpr.wait()
        for u in range(NSUB):
            accum(comm_f, 0, u, oc_f, 0, init=True)
            rd['f', 0, u] = start_hop(comm_f, 0, u, send_f, recv_f, right)
            accum(comm_r, 0, u, oc_r, NH, init=True)
            rd['r', 0, u] = start_hop(comm_r, 0, u, send_r, recv_r, left)
        cpf = load(oc_f, (me - 2) % N_DEV, load_f_sem)
        cpr = load(oc_r, (me + 2) % N_DEV, load_r_sem)

        for h in range(N_DEV - 1):
            final = h == N_DEV - 2
            rs = (h + 1) % 2
            cpf.wait()
            cpr.wait()
            for u in range(NSUB):
                rd['f', h, u].wait_recv()
                if h > 0:
                    rd['f', h - 1, u].wait_send()
                accum(comm_f, rs, u, oc_f, 0,
                      final_stage=stage_f if final else None)
                if not final:
                    rd['f', h + 1, u] = start_hop(
                        comm_f, h + 1, u, send_f, recv_f, right)
                rd['r', h, u].wait_recv()
                if h > 0:
                    rd['r', h - 1, u].wait_send()
                accum(comm_r, rs, u, oc_r, NH,
                      final_stage=stage_r if final else None)
                if not final:
                    rd['r', h + 1, u] = start_hop(
                        comm_r, h + 1, u, send_r, recv_r, left)
            if not final:
                cpf = load(oc_f, (me - h - 3) % N_DEV, load_f_sem)
                cpr = load(oc_r, (me + h + 3) % N_DEV, load_r_sem)

        cp0 = pltpu.make_async_copy(
            stage_f, out_hbm.at[:, :, pl.ds(0, NH)], store_sems.at[0])
        cp1 = pltpu.make_async_copy(
            stage_r, out_hbm.at[:, :, pl.ds(NH, NH)], store_sems.at[1])
        cp0.start()
        cp1.start()
        for u in range(NSUB):
            rd['f', N_DEV - 2, u].wait_send()
            rd['r', N_DEV - 2, u].wait_send()
        cp0.wait()
        cp1.wait()

    return pl.pallas_call(
        body,
        out_shape=jax.ShapeDtypeStruct((B, S_out, N), jnp.float32),
        in_specs=[pl.BlockSpec(memory_space=pl.ANY),
                  pl.BlockSpec(memory_space=pltpu.VMEM)],
        out_specs=pl.BlockSpec(memory_space=pl.ANY),
        scratch_shapes=[
            pltpu.VMEM((B, S_out, K), jnp.bfloat16),
            pltpu.VMEM((B, S_out, K), jnp.bfloat16),
            pltpu.VMEM((2, NSUB, B, S_out, SUB), jnp.bfloat16),
            pltpu.VMEM((2, NSUB, B, S_out, SUB), jnp.bfloat16),
            pltpu.VMEM((B, S_out, NH), jnp.float32),
            pltpu.VMEM((B, S_out, NH), jnp.float32),
            pltpu.SemaphoreType.DMA,
            pltpu.SemaphoreType.DMA,
            pltpu.SemaphoreType.DMA((2,)),
            pltpu.SemaphoreType.DMA((N_DEV - 1, NSUB)),
            pltpu.SemaphoreType.DMA((N_DEV - 1, NSUB)),
            pltpu.SemaphoreType.DMA((N_DEV - 1, NSUB)),
            pltpu.SemaphoreType.DMA((N_DEV - 1, NSUB)),
        ],
        compiler_params=pltpu.CompilerParams(
            collective_id=0,
            vmem_limit_bytes=46 * 1024 * 1024,
        ),
    )(O, Wo)


# device time: 184958 ns/iter; 3.4076x vs baseline; 1.0399x over previous
import jax
import jax.numpy as jnp
from jax import lax
from jax.experimental import pallas as pl
from jax.experimental.pallas import tpu as pltpu

N_DEV = 4
NSUB = 4


def kernel(O, Wo):
    B, S, Hl, D = O.shape
    K = Hl * D
    N = Wo.shape[1]
    S_out = S // N_DEV
    NH = N // 2
    SUB = NH // NSUB
    O = O.reshape(B, S, K).astype(jnp.bfloat16)
    Wo = Wo.astype(jnp.bfloat16)

    def body(o_hbm, w_ref, out_hbm,
             oc_f, oc_r, comm_f, comm_r, stage_f, stage_r,
             load_f_sem, load_r_sem, store_sems,
             send_f, recv_f, send_r, recv_r):
        me = lax.axis_index("i")
        left = (me - 1) % N_DEV
        right = (me + 1) % N_DEV

        barrier = pltpu.get_barrier_semaphore()
        for nbr in (left, right):
            pl.semaphore_signal(barrier, inc=1, device_id=(nbr,),
                                device_id_type=pl.DeviceIdType.MESH)
        pl.semaphore_wait(barrier, 2)

        def load(o_ref, c, sem):
            cp = pltpu.make_async_copy(
                o_hbm.at[:, pl.ds(c * S_out, S_out)], o_ref, sem)
            cp.start()
            return cp

        def start_hop(comm, h, u, ssem, rsem, dev):
            r = pltpu.make_async_remote_copy(
                src_ref=comm.at[h % 2, u],
                dst_ref=comm.at[(h + 1) % 2, u],
                send_sem=ssem.at[h, u], recv_sem=rsem.at[h, u],
                device_id=(dev,), device_id_type=pl.DeviceIdType.MESH)
            r.start()
            return r

        def accum(comm, slot, u, o_ref, col0, init=False, final_stage=None):
            for b in range(B):
                p = jnp.dot(o_ref[b],
                            w_ref[:, col0 + u * SUB:col0 + (u + 1) * SUB],
                            preferred_element_type=jnp.float32)
                if init:
                    comm[slot, u, b] = p.astype(jnp.bfloat16)
                elif final_stage is not None:
                    final_stage[b, :, u * SUB:(u + 1) * SUB] = (
                        comm[slot, u, b].astype(jnp.float32) + p)
                else:
                    comm[slot, u, b] = (
                        comm[slot, u, b].astype(jnp.float32) + p
                    ).astype(jnp.bfloat16)

        rd = {}
        cpf = load(oc_f, (me - 1) % N_DEV, load_f_sem)
        cpr = load(oc_r, (me + 1) % N_DEV, load_r_sem)
        cpf.wait()
        cpr.wait()
        for u in range(NSUB):
            accum(comm_f, 0, u, oc_f, 0, init=True)
            rd['f', 0, u] = start_hop(comm_f, 0, u, send_f, recv_f, right)
            accum(comm_r, 0, u, oc_r, NH, init=True)
            rd['r', 0, u] = start_hop(comm_r, 0, u, send_r, recv_r, left)
        cpf = load(oc_f, (me - 2) % N_DEV, load_f_sem)
        cpr = load(oc_r, (me + 2) % N_DEV, load_r_sem)

        for h in range(N_DEV - 1):
            final = h == N_DEV - 2
            rs = (h + 1) % 2
            cpf.wait()
            cpr.wait()
            stores = []
            for u in range(NSUB):
                rd['f', h, u].wait_recv()
                if h > 0:
                    rd['f', h - 1, u].wait_send()
                accum(comm_f, rs, u, oc_f, 0,
                      final_stage=stage_f if final else None)
                if not final:
                    rd['f', h + 1, u] = start_hop(
                        comm_f, h + 1, u, send_f, recv_f, right)
                else:
                    cp = pltpu.make_async_copy(
                        stage_f.at[:, :, pl.ds(u * SUB, SUB)],
                        out_hbm.at[:, :, pl.ds(u * SUB, SUB)],
                        store_sems.at[0, u])
                    cp.start()
                    stores.append(cp)
                rd['r', h, u].wait_recv()
                if h > 0:
                    rd['r', h - 1, u].wait_send()
                accum(comm_r, rs, u, oc_r, NH,
                      final_stage=stage_r if final else None)
                if not final:
                    rd['r', h + 1, u] = start_hop(
                        comm_r, h + 1, u, send_r, recv_r, left)
                else:
                    cp = pltpu.make_async_copy(
                        stage_r.at[:, :, pl.ds(u * SUB, SUB)],
                        out_hbm.at[:, :, pl.ds(NH + u * SUB, SUB)],
                        store_sems.at[1, u])
                    cp.start()
                    stores.append(cp)
            if not final:
                cpf = load(oc_f, (me - h - 3) % N_DEV, load_f_sem)
                cpr = load(oc_r, (me + h + 3) % N_DEV, load_r_sem)

        for u in range(NSUB):
            rd['f', N_DEV - 2, u].wait_send()
            rd['r', N_DEV - 2, u].wait_send()
        for cp in stores:
            cp.wait()

    return pl.pallas_call(
        body,
        out_shape=jax.ShapeDtypeStruct((B, S_out, N), jnp.float32),
        in_specs=[pl.BlockSpec(memory_space=pl.ANY),
                  pl.BlockSpec(memory_space=pltpu.VMEM)],
        out_specs=pl.BlockSpec(memory_space=pl.ANY),
        scratch_shapes=[
            pltpu.VMEM((B, S_out, K), jnp.bfloat16),
            pltpu.VMEM((B, S_out, K), jnp.bfloat16),
            pltpu.VMEM((2, NSUB, B, S_out, SUB), jnp.bfloat16),
            pltpu.VMEM((2, NSUB, B, S_out, SUB), jnp.bfloat16),
            pltpu.VMEM((B, S_out, NH), jnp.float32),
            pltpu.VMEM((B, S_out, NH), jnp.float32),
            pltpu.SemaphoreType.DMA,
            pltpu.SemaphoreType.DMA,
            pltpu.SemaphoreType.DMA((2, NSUB)),
            pltpu.SemaphoreType.DMA((N_DEV - 1, NSUB)),
            pltpu.SemaphoreType.DMA((N_DEV - 1, NSUB)),
            pltpu.SemaphoreType.DMA((N_DEV - 1, NSUB)),
            pltpu.SemaphoreType.DMA((N_DEV - 1, NSUB)),
        ],
        compiler_params=pltpu.CompilerParams(
            collective_id=0,
            vmem_limit_bytes=46 * 1024 * 1024,
        ),
    )(O, Wo)


# device time: 183710 ns/iter; 3.4308x vs baseline; 1.0068x over previous
import jax
import jax.numpy as jnp
from jax import lax
from jax.experimental import pallas as pl
from jax.experimental.pallas import tpu as pltpu

N_DEV = 4
NSUB = 4


def kernel(O, Wo):
    B, S, Hl, D = O.shape
    K = Hl * D
    N = Wo.shape[1]
    S_out = S // N_DEV
    NH = N // 2
    SUB = NH // NSUB
    O = O.reshape(B, S, K).astype(jnp.bfloat16)
    Wo = Wo.astype(jnp.bfloat16)

    def body(o_hbm, w_ref, out_hbm,
             oc_f, oc_r, comm_f, comm_r, stage_f, stage_r,
             load_f_sem, load_r_sem, store_sems,
             send_f, recv_f, send_r, recv_r):
        me = lax.axis_index("i")
        left = (me - 1) % N_DEV
        right = (me + 1) % N_DEV

        barrier = pltpu.get_barrier_semaphore()
        for nbr in (left, right):
            pl.semaphore_signal(barrier, inc=1, device_id=(nbr,),
                                device_id_type=pl.DeviceIdType.MESH)
        pl.semaphore_wait(barrier, 2)

        def load(o_ref, c, sem):
            cp = pltpu.make_async_copy(
                o_hbm.at[:, pl.ds(c * S_out, S_out)], o_ref, sem)
            cp.start()
            return cp

        def start_hop(comm, h, u, ssem, rsem, dev):
            r = pltpu.make_async_remote_copy(
                src_ref=comm.at[h % 2, u],
                dst_ref=comm.at[(h + 1) % 2, u],
                send_sem=ssem.at[h, u], recv_sem=rsem.at[h, u],
                device_id=(dev,), device_id_type=pl.DeviceIdType.MESH)
            r.start()
            return r

        def accum_init(comm, u, o_ref, col0):
            for b in range(B):
                p = jnp.dot(o_ref[b],
                            w_ref[:, col0 + u * SUB:col0 + (u + 1) * SUB],
                            preferred_element_type=jnp.float32)
                comm[0, u, b] = p.astype(jnp.bfloat16)

        def precompute(stage, o_ref, col0):
            for u in range(NSUB):
                for b in range(B):
                    stage[b, :, u * SUB:(u + 1) * SUB] = jnp.dot(
                        o_ref[b],
                        w_ref[:, col0 + u * SUB:col0 + (u + 1) * SUB],
                        preferred_element_type=jnp.float32)

        def add_round(comm, slot, u, stage):
            for b in range(B):
                comm[slot, u, b] = (
                    comm[slot, u, b].astype(jnp.float32)
                    + stage[b, :, u * SUB:(u + 1) * SUB]
                ).astype(jnp.bfloat16)

        def final_add(stage, comm, slot, u):
            for b in range(B):
                stage[b, :, u * SUB:(u + 1) * SUB] = (
                    stage[b, :, u * SUB:(u + 1) * SUB]
                    + comm[slot, u, b].astype(jnp.float32))

        rd = {}
        cpf = load(oc_f, (me - 1) % N_DEV, load_f_sem)
        cpr = load(oc_r, (me + 1) % N_DEV, load_r_sem)
        cpf.wait()
        cpr.wait()
        for u in range(NSUB):
            accum_init(comm_f, u, oc_f, 0)
            rd['f', 0, u] = start_hop(comm_f, 0, u, send_f, recv_f, right)
            accum_init(comm_r, u, oc_r, NH)
            rd['r', 0, u] = start_hop(comm_r, 0, u, send_r, recv_r, left)
        cpf = load(oc_f, (me - 2) % N_DEV, load_f_sem)
        cpr = load(oc_r, (me + 2) % N_DEV, load_r_sem)
        cpf.wait()
        cpr.wait()
        precompute(stage_f, oc_f, 0)
        precompute(stage_r, oc_r, NH)

        for h in range(N_DEV - 1):
            final = h == N_DEV - 2
            rs = (h + 1) % 2
            if not final:
                cpf = load(oc_f, (me - h - 3) % N_DEV, load_f_sem)
                cpr = load(oc_r, (me + h + 3) % N_DEV, load_r_sem)
            stores = []
            for u in range(NSUB):
                rd['f', h, u].wait_recv()
                if h > 0:
                    rd['f', h - 1, u].wait_send()
                if not final:
                    add_round(comm_f, rs, u, stage_f)
                    rd['f', h + 1, u] = start_hop(
                        comm_f, h + 1, u, send_f, recv_f, right)
                else:
                    final_add(stage_f, comm_f, rs, u)
                    cp = pltpu.make_async_copy(
                        stage_f.at[:, :, pl.ds(u * SUB, SUB)],
                        out_hbm.at[:, :, pl.ds(u * SUB, SUB)],
                        store_sems.at[0, u])
                    cp.start()
                    stores.append(cp)
                rd['r', h, u].wait_recv()
                if h > 0:
                    rd['r', h - 1, u].wait_send()
                if not final:
                    add_round(comm_r, rs, u, stage_r)
                    rd['r', h + 1, u] = start_hop(
                        comm_r, h + 1, u, send_r, recv_r, left)
                else:
                    final_add(stage_r, comm_r, rs, u)
                    cp = pltpu.make_async_copy(
                        stage_r.at[:, :, pl.ds(u * SUB, SUB)],
                        out_hbm.at[:, :, pl.ds(NH + u * SUB, SUB)],
                        store_sems.at[1, u])
                    cp.start()
                    stores.append(cp)
            if not final:
                cpf.wait()
                cpr.wait()
                precompute(stage_f, oc_f, 0)
                precompute(stage_r, oc_r, NH)

        for u in range(NSUB):
            rd['f', N_DEV - 2, u].wait_send()
            rd['r', N_DEV - 2, u].wait_send()
        for cp in stores:
            cp.wait()

    return pl.pallas_call(
        body,
        out_shape=jax.ShapeDtypeStruct((B, S_out, N), jnp.float32),
        in_specs=[pl.BlockSpec(memory_space=pl.ANY),
                  pl.BlockSpec(memory_space=pltpu.VMEM)],
        out_specs=pl.BlockSpec(memory_space=pl.ANY),
        scratch_shapes=[
            pltpu.VMEM((B, S_out, K), jnp.bfloat16),
            pltpu.VMEM((B, S_out, K), jnp.bfloat16),
            pltpu.VMEM((2, NSUB, B, S_out, SUB), jnp.bfloat16),
            pltpu.VMEM((2, NSUB, B, S_out, SUB), jnp.bfloat16),
            pltpu.VMEM((B, S_out, NH), jnp.float32),
            pltpu.VMEM((B, S_out, NH), jnp.float32),
            pltpu.SemaphoreType.DMA,
            pltpu.SemaphoreType.DMA,
            pltpu.SemaphoreType.DMA((2, NSUB)),
            pltpu.SemaphoreType.DMA((N_DEV - 1, NSUB)),
            pltpu.SemaphoreType.DMA((N_DEV - 1, NSUB)),
            pltpu.SemaphoreType.DMA((N_DEV - 1, NSUB)),
            pltpu.SemaphoreType.DMA((N_DEV - 1, NSUB)),
        ],
        compiler_params=pltpu.CompilerParams(
            collective_id=0,
            vmem_limit_bytes=46 * 1024 * 1024,
        ),
    )(O, Wo)


# device time: 173578 ns/iter; 3.6310x vs baseline; 1.0584x over previous
import jax
import jax.numpy as jnp
from jax import lax
from jax.experimental import pallas as pl
from jax.experimental.pallas import tpu as pltpu

N_DEV = 4
NSUB = 4


def kernel(O, Wo):
    B, S, Hl, D = O.shape
    K = Hl * D
    N = Wo.shape[1]
    S_out = S // N_DEV
    NH = N // 2
    SUB = NH // NSUB
    O = jnp.transpose(O, (0, 2, 3, 1)).reshape(B, K, S)
    Wo = Wo.astype(jnp.bfloat16)
    DN = (((0,), (0,)), ((), ()))

    def body(o_hbm, w_ref, out_hbm,
             oc_f, oc_r, ob_f, ob_r, comm_f, comm_r, stage_f, stage_r,
             load_f_sem, load_r_sem, store_sems,
             send_f, recv_f, send_r, recv_r):
        me = lax.axis_index("i")
        left = (me - 1) % N_DEV
        right = (me + 1) % N_DEV

        barrier = pltpu.get_barrier_semaphore()
        for nbr in (left, right):
            pl.semaphore_signal(barrier, inc=1, device_id=(nbr,),
                                device_id_type=pl.DeviceIdType.MESH)
        pl.semaphore_wait(barrier, 2)

        def load(o_ref, c, sem):
            cp = pltpu.make_async_copy(
                o_hbm.at[:, :, pl.ds(c * S_out, S_out)], o_ref, sem)
            cp.start()
            return cp

        def to_bf16(ob_ref, oc_ref):
            for b in range(B):
                ob_ref[b] = oc_ref[b].astype(jnp.bfloat16)

        def start_hop(comm, h, u, ssem, rsem, dev):
            r = pltpu.make_async_remote_copy(
                src_ref=comm.at[h % 2, u],
                dst_ref=comm.at[(h + 1) % 2, u],
                send_sem=ssem.at[h, u], recv_sem=rsem.at[h, u],
                device_id=(dev,), device_id_type=pl.DeviceIdType.MESH)
            r.start()
            return r

        def accum_init(comm, u, o_ref, col0):
            for b in range(B):
                p = lax.dot_general(
                    o_ref[b], w_ref[:, col0 + u * SUB:col0 + (u + 1) * SUB],
                    DN, preferred_element_type=jnp.float32)
                comm[0, u, b] = p.astype(jnp.bfloat16)

        def precompute(stage, o_ref, col0):
            for u in range(NSUB):
                for b in range(B):
                    stage[b, :, u * SUB:(u + 1) * SUB] = lax.dot_general(
                        o_ref[b],
                        w_ref[:, col0 + u * SUB:col0 + (u + 1) * SUB],
                        DN, preferred_element_type=jnp.float32)

        def add_round(comm, slot, u, stage):
            for b in range(B):
                comm[slot, u, b] = (
                    comm[slot, u, b].astype(jnp.float32)
                    + stage[b, :, u * SUB:(u + 1) * SUB]
                ).astype(jnp.bfloat16)

        def final_add(stage, comm, slot, u):
            for b in range(B):
                stage[b, :, u * SUB:(u + 1) * SUB] = (
                    stage[b, :, u * SUB:(u + 1) * SUB]
                    + comm[slot, u, b].astype(jnp.float32))

        rd = {}
        cpf = load(oc_f, (me - 1) % N_DEV, load_f_sem)
        cpr = load(oc_r, (me + 1) % N_DEV, load_r_sem)
        cpf.wait()
        to_bf16(ob_f, oc_f)
        cpr.wait()
        to_bf16(ob_r, oc_r)
        for u in range(NSUB):
            accum_init(comm_f, u, ob_f, 0)
            rd['f', 0, u] = start_hop(comm_f, 0, u, send_f, recv_f, right)
            accum_init(comm_r, u, ob_r, NH)
            rd['r', 0, u] = start_hop(comm_r, 0, u, send_r, recv_r, left)
        cpf = load(oc_f, (me - 2) % N_DEV, load_f_sem)
        cpr = load(oc_r, (me + 2) % N_DEV, load_r_sem)
        cpf.wait()
        to_bf16(ob_f, oc_f)
        cpr.wait()
        to_bf16(ob_r, oc_r)
        precompute(stage_f, ob_f, 0)
        precompute(stage_r, ob_r, NH)

        for h in range(N_DEV - 1):
            final = h == N_DEV - 2
            rs = (h + 1) % 2
            if not final:
                cpf = load(oc_f, (me - h - 3) % N_DEV, load_f_sem)
                cpr = load(oc_r, (me + h + 3) % N_DEV, load_r_sem)
            stores = []
            for u in range(NSUB):
                rd['f', h, u].wait_recv()
                if h > 0:
                    rd['f', h - 1, u].wait_send()
                if not final:
                    add_round(comm_f, rs, u, stage_f)
                    rd['f', h + 1, u] = start_hop(
                        comm_f, h + 1, u, send_f, recv_f, right)
                else:
                    final_add(stage_f, comm_f, rs, u)
                    cp = pltpu.make_async_copy(
                        stage_f.at[:, :, pl.ds(u * SUB, SUB)],
                        out_hbm.at[:, :, pl.ds(u * SUB, SUB)],
                        store_sems.at[0, u])
                    cp.start()
                    stores.append(cp)
                rd['r', h, u].wait_recv()
                if h > 0:
                    rd['r', h - 1, u].wait_send()
                if not final:
                    add_round(comm_r, rs, u, stage_r)
                    rd['r', h + 1, u] = start_hop(
                        comm_r, h + 1, u, send_r, recv_r, left)
                else:
                    final_add(stage_r, comm_r, rs, u)
                    cp = pltpu.make_async_copy(
                        stage_r.at[:, :, pl.ds(u * SUB, SUB)],
                        out_hbm.at[:, :, pl.ds(NH + u * SUB, SUB)],
                        store_sems.at[1, u])
                    cp.start()
                    stores.append(cp)
            if not final:
                cpf.wait()
                to_bf16(ob_f, oc_f)
                cpr.wait()
                to_bf16(ob_r, oc_r)
                precompute(stage_f, ob_f, 0)
                precompute(stage_r, ob_r, NH)

        for u in range(NSUB):
            rd['f', N_DEV - 2, u].wait_send()
            rd['r', N_DEV - 2, u].wait_send()
        for cp in stores:
            cp.wait()

    return pl.pallas_call(
        body,
        out_shape=jax.ShapeDtypeStruct((B, S_out, N), jnp.float32),
        in_specs=[pl.BlockSpec(memory_space=pl.ANY),
                  pl.BlockSpec(memory_space=pltpu.VMEM)],
        out_specs=pl.BlockSpec(memory_space=pl.ANY),
        scratch_shapes=[
            pltpu.VMEM((B, K, S_out), jnp.float32),
            pltpu.VMEM((B, K, S_out), jnp.float32),
            pltpu.VMEM((B, K, S_out), jnp.bfloat16),
            pltpu.VMEM((B, K, S_out), jnp.bfloat16),
            pltpu.VMEM((2, NSUB, B, S_out, SUB), jnp.bfloat16),
            pltpu.VMEM((2, NSUB, B, S_out, SUB), jnp.bfloat16),
            pltpu.VMEM((B, S_out, NH), jnp.float32),
            pltpu.VMEM((B, S_out, NH), jnp.float32),
            pltpu.SemaphoreType.DMA,
            pltpu.SemaphoreType.DMA,
            pltpu.SemaphoreType.DMA((2, NSUB)),
            pltpu.SemaphoreType.DMA((N_DEV - 1, NSUB)),
            pltpu.SemaphoreType.DMA((N_DEV - 1, NSUB)),
            pltpu.SemaphoreType.DMA((N_DEV - 1, NSUB)),
            pltpu.SemaphoreType.DMA((N_DEV - 1, NSUB)),
        ],
        compiler_params=pltpu.CompilerParams(
            collective_id=0,
            vmem_limit_bytes=55 * 1024 * 1024,
        ),
    )(O, Wo)
